# Initial kernel scaffold; baseline (speedup 1.0000x reference)
import functools

import jax
import jax.numpy as jnp
from jax import lax
from jax.experimental import pallas as pl
from jax.experimental.pallas import tpu as pltpu
from jax.experimental.pallas import tpu_sc as plsc

N = 10000
E = 160000
NC = 1000
C = 8
DOP = 140
DEMB = 32
DCFG = 18
H = 32
NUM_OPS = 120

NP = 10240
EP = 163840
ECH = EP // 128
NCP = 1024
BN = 1024
NBLK = NP // BN
ROWS_T = NP // 16
CH_T = ECH // 16
CH_T2 = ECH // 32

_mesh = plsc.VectorSubcoreMesh(core_axis_name="c", subcore_axis_name="s")


def _leaky(x):
    return jnp.where(x > 0, x, 0.2 * x)


def _sc_build(src_j, dst_j, cfgf, cfgdst_j, zerov, onesv, zer64,
              deg_out, cfg_out,
              sdeg, scfg, vz, vo, vz2, sidx, didx, ci, crows):
    c = lax.axis_index("c")
    s = lax.axis_index("s")
    pltpu.sync_copy(zerov, vz)
    pltpu.sync_copy(onesv, vo)
    pltpu.sync_copy(zer64, vz2)
    r0 = s * ROWS_T
    for r in range(ROWS_T // 128):
        pltpu.sync_copy(vz, sdeg.at[pl.ds(r0 + r * 128, 128)])
    for r in range(ROWS_T // 64):
        pltpu.sync_copy(vz2, scfg.at[pl.ds(r0 + r * 64, 64)])
    plsc.subcore_barrier()

    base = (c * 16 + s) * CH_T2
    pltpu.sync_copy(src_j.at[pl.ds(base, CH_T2)], sidx)
    pltpu.sync_copy(dst_j.at[pl.ds(base, CH_T2)], didx)

    def dbody(j, _):
        pltpu.sync_copy(vo, sdeg.at[sidx.at[j]], add=True)
        pltpu.sync_copy(vo, sdeg.at[didx.at[j]], add=True)
        return 0

    lax.fori_loop(0, CH_T2, dbody, 0)

    pltpu.sync_copy(cfgdst_j.at[s], ci)
    pltpu.sync_copy(cfgf.at[pl.ds(s * 64, 64), pl.ds(c * 72, 72)], crows)
    pltpu.sync_copy(crows, scfg.at[ci], add=True)
    plsc.subcore_barrier()

    pltpu.sync_copy(sdeg.at[pl.ds(r0, ROWS_T)], deg_out.at[c, pl.ds(r0, ROWS_T)])
    pltpu.sync_copy(scfg.at[pl.ds(r0, ROWS_T)], cfg_out.at[c, pl.ds(r0, ROWS_T)])


def _run_sc_build(src_j, dst_j, cfgf, cfgdst_j):
    zerov = jnp.zeros((128,), jnp.float32)
    onesv = jnp.ones((128,), jnp.float32)
    zer64 = jnp.zeros((64, 72), jnp.float32)
    f = pl.kernel(
        _sc_build,
        out_type=(
            jax.ShapeDtypeStruct((2, NP), jnp.float32),
            jax.ShapeDtypeStruct((2, NP, 72), jnp.float32),
        ),
        mesh=_mesh,
        scratch_types=[
            pltpu.VMEM_SHARED((NP,), jnp.float32),
            pltpu.VMEM_SHARED((NP, 72), jnp.float32),
            pltpu.VMEM((128,), jnp.float32),
            pltpu.VMEM((128,), jnp.float32),
            pltpu.VMEM((64, 72), jnp.float32),
            pltpu.VMEM((CH_T2, 128), jnp.int32),
            pltpu.VMEM((CH_T2, 128), jnp.int32),
            pltpu.VMEM((64,), jnp.int32),
            pltpu.VMEM((64, 72), jnp.float32),
        ],
    )
    return f(src_j, dst_j, cfgf, cfgdst_j, zerov, onesv, zer64)


def _sc_agg(tab, src_j, dst_j, out,
            acc, sidx, didx, sidxo, didxo, buf0, buf1, sem0, sem1):
    c = lax.axis_index("c")
    s = lax.axis_index("s")
    coff = c * NP
    r0 = s * ROWS_T

    pltpu.sync_copy(src_j.at[pl.ds(s * CH_T, CH_T)], sidx)
    pltpu.sync_copy(dst_j.at[pl.ds(s * CH_T, CH_T)], didx)

    def obody(j, _):
        for v in range(8):
            sl = pl.ds(v * 16, 16)
            sidxo[j, sl] = sidx[j, sl] + coff
            didxo[j, sl] = didx[j, sl] + coff
        return 0

    lax.fori_loop(0, CH_T, obody, 0)

    pltpu.sync_copy(tab.at[pl.ds(coff + r0, ROWS_T)], acc.at[pl.ds(r0, ROWS_T)])
    plsc.subcore_barrier()

    def run_dir(gidx, scidx):
        def issue(j, buf, sem):
            pltpu.async_copy(tab.at[gidx.at[j]], buf, sem)

        def drain(j, buf, sem):
            pltpu.make_async_copy(tab.at[gidx.at[j]], buf, sem).wait()
            pltpu.sync_copy(buf, acc.at[scidx.at[j]], add=True)

        issue(0, buf0, sem0)
        issue(1, buf1, sem1)

        def body(i, _):
            j = 2 * i
            drain(j, buf0, sem0)
            issue(j + 2, buf0, sem0)
            drain(j + 1, buf1, sem1)
            issue(j + 3, buf1, sem1)
            return 0

        lax.fori_loop(0, CH_T // 2 - 1, body, 0)
        drain(CH_T - 2, buf0, sem0)
        drain(CH_T - 1, buf1, sem1)

    run_dir(sidxo, didx)
    run_dir(didxo, sidx)
    plsc.subcore_barrier()

    pltpu.sync_copy(acc.at[pl.ds(r0, ROWS_T)], out.at[c, pl.ds(r0, ROWS_T)])


def _run_sc_agg(tab, src_j, dst_j, width):
    f = pl.kernel(
        _sc_agg,
        out_type=jax.ShapeDtypeStruct((2, NP, width), jnp.float32),
        mesh=_mesh,
        scratch_types=[
            pltpu.VMEM_SHARED((NP, width), jnp.float32),
            pltpu.VMEM((CH_T, 128), jnp.int32),
            pltpu.VMEM((CH_T, 128), jnp.int32),
            pltpu.VMEM((CH_T, 128), jnp.int32),
            pltpu.VMEM((CH_T, 128), jnp.int32),
            pltpu.VMEM((128, width), jnp.float32),
            pltpu.VMEM((128, width), jnp.float32),
            pltpu.SemaphoreType.DMA,
            pltpu.SemaphoreType.DMA,
        ],
    )
    return f(tab, src_j, dst_j)


def _sc_final(x2, cfgdst_j, out,
              spart, ci, cio, cbuf, obuf, pbuf, rbuf, sem):
    c = lax.axis_index("c")
    s = lax.axis_index("s")
    coff = c * NP

    pltpu.sync_copy(cfgdst_j.at[s], ci)
    for v in range(4):
        sl = pl.ds(v * 16, 16)
        cio[sl] = ci[sl] + coff
    pltpu.async_copy(x2.at[cio], cbuf, sem).wait()
    cacc = [jnp.zeros((16,), jnp.float32) for _ in range(8)]
    for r in range(64):
        for v in range(8):
            cacc[v] = cacc[v] + cbuf[r, pl.ds(v * 16, 16)]
    for v in range(8):
        pbuf[1, pl.ds(v * 16, 16)] = cacc[v]

    def kbody(k, carry):
        pltpu.sync_copy(x2.at[pl.ds(coff + s * ROWS_T + k * 64, 64)], obuf)
        vs = list(carry)
        for r in range(64):
            for v in range(8):
                vs[v] = vs[v] + obuf[r, pl.ds(v * 16, 16)]
        return tuple(vs)

    oacc = lax.fori_loop(0, ROWS_T // 64, kbody,
                         tuple(jnp.zeros((16,), jnp.float32) for _ in range(8)))
    for v in range(8):
        pbuf[0, pl.ds(v * 16, 16)] = oacc[v]

    pltpu.sync_copy(pbuf, spart.at[s])
    plsc.subcore_barrier()

    @pl.when(s == 0)
    def _():
        pltpu.sync_copy(spart, rbuf)
        for g in range(2):
            for v in range(8):
                t = jnp.zeros((16,), jnp.float32)
                for w in range(16):
                    t = t + rbuf[w, g, pl.ds(v * 16, 16)]
                pbuf[g, pl.ds(v * 16, 16)] = t
        pltpu.sync_copy(pbuf, out.at[c])


def _run_sc_final(x2, cfgdst_j):
    f = pl.kernel(
        _sc_final,
        out_type=jax.ShapeDtypeStruct((2, 2, 128), jnp.float32),
        mesh=_mesh,
        scratch_types=[
            pltpu.VMEM_SHARED((16, 2, 128), jnp.float32),
            pltpu.VMEM((64,), jnp.int32),
            pltpu.VMEM((64,), jnp.int32),
            pltpu.VMEM((64, 128), jnp.float32),
            pltpu.VMEM((64, 128), jnp.float32),
            pltpu.VMEM((2, 128), jnp.float32),
            pltpu.VMEM((16, 2, 128), jnp.float32),
            pltpu.SemaphoreType.DMA,
        ],
    )
    return f(x2, cfgdst_j)


def _tc_prenet(nf_ref, ids_ref, d0_ref, d1_ref, cfg_ref,
               emb_ref, w1_ref, b1_ref, w2_ref, b2_ref,
               ys_ref, csc_ref):
    i = pl.program_id(0)
    rows = i * BN + lax.broadcasted_iota(jnp.int32, (BN, 1), 0)
    msk = (rows < N).astype(jnp.float32)
    deg = 1.0 + d0_ref[...] + d1_ref[...]
    inv = lax.rsqrt(deg)[:, None] * msk

    w1 = w1_ref[...]
    ew = jnp.dot(emb_ref[...], w1[158:190], preferred_element_type=jnp.float32)
    oh = (ids_ref[...][:, None] ==
          lax.broadcasted_iota(jnp.int32, (BN, NUM_OPS), 1)).astype(jnp.float32)
    z = (jnp.dot(nf_ref[...], w1[18:158], preferred_element_type=jnp.float32)
         + jnp.dot(oh, ew, preferred_element_type=jnp.float32) + b1_ref[...])
    w1c = w1[0:18]
    w2 = w2_ref[...]
    b2 = b2_ref[...]
    for h in range(2):
        cfg = cfg_ref[h]
        csc_ref[h] = cfg * (100.0 * inv)
        for j in range(4):
            zc = jnp.dot(100.0 * cfg[:, j * 18:(j + 1) * 18], w1c,
                         preferred_element_type=jnp.float32)
            x = _leaky(jnp.dot(_leaky(z + zc), w2,
                               preferred_element_type=jnp.float32) + b2)
            ys_ref[h, :, j * 32:(j + 1) * 32] = x * inv


def _run_tc_prenet(node_feats, op_ids, d0, d1, cfgacc, op_emb, w1, b1, w2, b2):
    return pl.pallas_call(
        _tc_prenet,
        grid=(NBLK,),
        in_specs=[
            pl.BlockSpec((BN, DOP), lambda i: (i, 0)),
            pl.BlockSpec((BN,), lambda i: (i,)),
            pl.BlockSpec((BN,), lambda i: (i,)),
            pl.BlockSpec((BN,), lambda i: (i,)),
            pl.BlockSpec((2, BN, 72), lambda i: (0, i, 0)),
            pl.BlockSpec((NUM_OPS, DEMB), lambda i: (0, 0)),
            pl.BlockSpec((190, H), lambda i: (0, 0)),
            pl.BlockSpec((H,), lambda i: (0,)),
            pl.BlockSpec((H, H), lambda i: (0, 0)),
            pl.BlockSpec((H,), lambda i: (0,)),
        ],
        out_specs=[
            pl.BlockSpec((2, BN, 128), lambda i: (0, i, 0)),
            pl.BlockSpec((2, BN, 72), lambda i: (0, i, 0)),
        ],
        out_shape=[
            jax.ShapeDtypeStruct((2, NP, 128), jnp.float32),
            jax.ShapeDtypeStruct((2, NP, 72), jnp.float32),
        ],
    )(node_feats, op_ids, d0, d1, cfgacc, op_emb, w1, b1, w2, b2)


def _tc_layer(final, ys_ref, sx_ref, sc_ref, d0_ref, d1_ref,
              w1_ref, b1_ref, w2_ref, b2_ref, out_ref):
    i = pl.program_id(0)
    rows = i * BN + lax.broadcasted_iota(jnp.int32, (BN, 1), 0)
    msk = (rows < N).astype(jnp.float32)
    deg = 1.0 + d0_ref[...] + d1_ref[...]
    inv = lax.rsqrt(deg)[:, None]
    sq = jnp.sqrt(deg)[:, None]
    oscale = msk if final else inv * msk

    w1 = w1_ref[...]
    w1c = w1[0:18]
    w1x = w1[18:50]
    b1 = b1_ref[...]
    w2 = w2_ref[...]
    b2 = b2_ref[...]
    for h in range(2):
        x = ys_ref[h] * sq
        aggx = sx_ref[h] * inv
        aggc = sc_ref[h] * inv
        for j in range(4):
            pre = (jnp.dot(aggc[:, j * 18:(j + 1) * 18], w1c,
                           preferred_element_type=jnp.float32)
                   + jnp.dot(aggx[:, j * 32:(j + 1) * 32], w1x,
                             preferred_element_type=jnp.float32) + b1)
            y = _leaky(jnp.dot(_leaky(pre), w2,
                               preferred_element_type=jnp.float32) + b2)
            xn = x[:, j * 32:(j + 1) * 32] + y
            out_ref[h, :, j * 32:(j + 1) * 32] = xn * oscale


def _run_tc_layer(ys, sx, scfg, d0, d1, w1, b1, w2, b2, final):
    return pl.pallas_call(
        functools.partial(_tc_layer, final),
        grid=(NBLK,),
        in_specs=[
            pl.BlockSpec((2, BN, 128), lambda i: (0, i, 0)),
            pl.BlockSpec((2, BN, 128), lambda i: (0, i, 0)),
            pl.BlockSpec((2, BN, 72), lambda i: (0, i, 0)),
            pl.BlockSpec((BN,), lambda i: (i,)),
            pl.BlockSpec((BN,), lambda i: (i,)),
            pl.BlockSpec((50, H), lambda i: (0, 0)),
            pl.BlockSpec((H,), lambda i: (0,)),
            pl.BlockSpec((H, H), lambda i: (0, 0)),
            pl.BlockSpec((H,), lambda i: (0,)),
        ],
        out_specs=[pl.BlockSpec((2, BN, 128), lambda i: (0, i, 0))],
        out_shape=[jax.ShapeDtypeStruct((2, NP, 128), jnp.float32)],
    )(ys, sx, scfg, d0, d1, w1, b1, w2, b2)[0]


def _tc_postnet(op_ref, cfg_ref, w1_ref, w2_ref, out_ref):
    def l2n(v):
        return v * lax.rsqrt(jnp.maximum(jnp.sum(v * v, axis=-1, keepdims=True),
                                         1e-12))

    op_sum = op_ref[...]
    cfg_sum = cfg_ref[...]
    feat = jnp.concatenate([op_sum / float(N), l2n(op_sum), l2n(cfg_sum)],
                           axis=-1)
    r = jnp.dot(_leaky(jnp.dot(feat, w1_ref[...],
                               preferred_element_type=jnp.float32)),
                w2_ref[...], preferred_element_type=jnp.float32)
    out_ref[...] = r


def _run_tc_postnet(op_sum, cfg_sum, w1, w2):
    return pl.pallas_call(
        _tc_postnet,
        out_shape=jax.ShapeDtypeStruct((C, 1), jnp.float32),
    )(op_sum, cfg_sum, w1, w2)


def kernel(node_feats, config_feats, op_emb, prenet_w1, prenet_b1, prenet_w2,
           prenet_b2, gc0_w1, gc0_b1, gc0_w2, gc0_b2, gc1_w1, gc1_b1, gc1_w2,
           gc1_b2, postnet_w1, postnet_w2, op_ids, feed_src, feed_dst,
           cfg_src, cfg_dst):
    del cfg_src

    pad_e = (N + (jnp.arange(EP - E, dtype=jnp.int32) % (NP - N))).astype(jnp.int32)
    src_j = jnp.concatenate([feed_src, pad_e]).reshape(ECH, 128)
    dst_j = jnp.concatenate([feed_dst, pad_e]).reshape(ECH, 128)
    pad_c = (N + (jnp.arange(NCP - NC, dtype=jnp.int32) % (NP - N))).astype(jnp.int32)
    cfgdst_j = jnp.concatenate([cfg_dst, pad_c]).reshape(16, 64)
    cfgf = jnp.concatenate(
        [config_feats.reshape(NC, 144),
         jnp.zeros((NCP - NC, 144), jnp.float32)])

    deg2, cfgacc = _run_sc_build(src_j, dst_j, cfgf, cfgdst_j)
    d0, d1 = deg2[0], deg2[1]

    ys0, csc = _run_tc_prenet(node_feats, op_ids, d0, d1, cfgacc, op_emb,
                              prenet_w1, prenet_b1, prenet_w2, prenet_b2)

    s_cfg = _run_sc_agg(csc.reshape(2 * NP, 72), src_j, dst_j, 72)

    sx0 = _run_sc_agg(ys0.reshape(2 * NP, 128), src_j, dst_j, 128)
    ys1 = _run_tc_layer(ys0, sx0, s_cfg, d0, d1, gc0_w1, gc0_b1, gc0_w2,
                        gc0_b2, final=False)

    sx1 = _run_sc_agg(ys1.reshape(2 * NP, 128), src_j, dst_j, 128)
    x2 = _run_tc_layer(ys1, sx1, s_cfg, d0, d1, gc1_w1, gc1_b1, gc1_w2,
                       gc1_b2, final=True)

    sums = _run_sc_final(x2.reshape(2 * NP, 128), cfgdst_j)
    op_sum = sums[:, 0, :].reshape(C, H)
    cfg_sum = sums[:, 1, :].reshape(C, H)

    out = _run_tc_postnet(op_sum, cfg_sum, postnet_w1, postnet_w2)
    return out.reshape(1, C)


# speedup vs baseline: 79.4601x; 79.4601x over previous
import functools

import jax
import jax.numpy as jnp
from jax import lax
from jax.experimental import pallas as pl
from jax.experimental.pallas import tpu as pltpu
from jax.experimental.pallas import tpu_sc as plsc

N = 10000
E = 160000
NC = 1000
C = 8
DOP = 140
DEMB = 32
DCFG = 18
H = 32
NUM_OPS = 120

NP = 10240
EP = 163840
ECH = EP // 128
ECH32 = EP // 32
CH32_T = ECH32 // 16
NCP = 1024
BN = 1024
NBLK = NP // BN
ROWS_T = NP // 16
CH_T = ECH // 16
CH_T2 = ECH // 32

_mesh = plsc.VectorSubcoreMesh(core_axis_name="c", subcore_axis_name="s")


def _leaky(x):
    return jnp.where(x > 0, x, 0.2 * x)


def _sc_build(src_j, dst_j, cfgf, cfgdst_j, zerov, onesv, zer64,
              deg_out, cfg_out,
              sdeg, scfg, vz, vo, sidx, didx, ci, crows):
    c = lax.axis_index("c")
    s = lax.axis_index("s")
    pltpu.sync_copy(zerov, vz)
    pltpu.sync_copy(onesv, vo)
    pltpu.sync_copy(zer64, crows)
    r0 = s * ROWS_T
    for r in range(ROWS_T // 128):
        pltpu.sync_copy(vz, sdeg.at[pl.ds(r0 + r * 128, 128)])
    for r in range(ROWS_T // 64):
        pltpu.sync_copy(crows, scfg.at[pl.ds(r0 + r * 64, 64)])
    plsc.subcore_barrier()

    base = (c * 16 + s) * CH_T2
    pltpu.sync_copy(src_j.at[pl.ds(base, CH_T2)], sidx)
    pltpu.sync_copy(dst_j.at[pl.ds(base, CH_T2)], didx)

    for j in range(CH_T2):
        pltpu.sync_copy(vo, sdeg.at[sidx.at[j]], add=True)
        pltpu.sync_copy(vo, sdeg.at[didx.at[j]], add=True)

    pltpu.sync_copy(cfgdst_j.at[s], ci)
    pltpu.sync_copy(cfgf.at[c, pl.ds(s * 64, 64)], crows)
    pltpu.sync_copy(crows, scfg.at[ci], add=True)
    plsc.subcore_barrier()

    pltpu.sync_copy(sdeg.at[pl.ds(r0, ROWS_T)], deg_out.at[c, pl.ds(r0, ROWS_T)])
    for r in range(ROWS_T // 64):
        pltpu.sync_copy(scfg.at[pl.ds(r0 + r * 64, 64)],
                        cfg_out.at[c, pl.ds(r0 + r * 64, 64)])


def _run_sc_build(src_j, dst_j, cfgf, cfgdst_j):
    zerov = jnp.zeros((128,), jnp.float32)
    onesv = jnp.ones((128,), jnp.float32)
    zer64 = jnp.zeros((64, 128), jnp.float32)
    f = pl.kernel(
        _sc_build,
        out_type=(
            jax.ShapeDtypeStruct((2, NP), jnp.float32),
            jax.ShapeDtypeStruct((2, NP, 128), jnp.float32),
        ),
        mesh=_mesh,
        scratch_types=[
            pltpu.VMEM_SHARED((NP,), jnp.float32),
            pltpu.VMEM_SHARED((NP, 128), jnp.float32),
            pltpu.VMEM((128,), jnp.float32),
            pltpu.VMEM((128,), jnp.float32),
            pltpu.VMEM((CH_T2, 128), jnp.int32),
            pltpu.VMEM((CH_T2, 128), jnp.int32),
            pltpu.VMEM((64,), jnp.int32),
            pltpu.VMEM((64, 128), jnp.float32),
        ],
    )
    return f(src_j, dst_j, cfgf, cfgdst_j, zerov, onesv, zer64)


def _sc_agg(tab, src_j, dst_j, out,
            acc, gj, sj, buf0, buf1, sem0, sem1):
    c = lax.axis_index("c")
    s = lax.axis_index("s")
    coff = c * NP
    r0 = s * ROWS_T
    row0 = s * CH_T

    def ibody(r, _):
        pltpu.sync_copy(tab.at[pl.ds(coff + r0 + r * 128, 128)], buf0)
        pltpu.sync_copy(buf0, acc.at[pl.ds(r0 + r * 128, 128)])
        return 0

    lax.fori_loop(0, ROWS_T // 128, ibody, 0)
    plsc.subcore_barrier()

    def run_dir(gsel, ssel):
        def issue(j, buf, sem):
            pltpu.async_copy(tab.at[gj.at[j]], buf, sem)

        def drain(j, buf, sem):
            pltpu.make_async_copy(tab.at[gj.at[j]], buf, sem).wait()
            pltpu.sync_copy(buf, acc.at[sj.at[j]], add=True)

        def block(b, _):
            pltpu.sync_copy(gsel.at[pl.ds(row0 + b * 8, 8)], gj)
            pltpu.sync_copy(ssel.at[pl.ds(row0 + b * 8, 8)], sj)
            for r in range(8):
                for v in range(8):
                    sl = pl.ds(v * 16, 16)
                    gj[r, sl] = gj[r, sl] + coff
            issue(0, buf0, sem0)
            issue(1, buf1, sem1)
            for j in range(6):
                buf, sem = (buf0, sem0) if j % 2 == 0 else (buf1, sem1)
                drain(j, buf, sem)
                issue(j + 2, buf, sem)
            drain(6, buf0, sem0)
            drain(7, buf1, sem1)
            return 0

        lax.fori_loop(0, CH_T // 8, block, 0)

    run_dir(src_j, dst_j)
    run_dir(dst_j, src_j)
    plsc.subcore_barrier()

    def wbody(r, _):
        pltpu.sync_copy(acc.at[pl.ds(r0 + r * 128, 128)], buf1)
        pltpu.sync_copy(buf1, out.at[c, pl.ds(r0 + r * 128, 128)])
        return 0

    lax.fori_loop(0, ROWS_T // 128, wbody, 0)


def _run_sc_agg(tab, src_j, dst_j, width):
    f = pl.kernel(
        _sc_agg,
        out_type=jax.ShapeDtypeStruct((2, NP, width), jnp.float32),
        mesh=_mesh,
        scratch_types=[
            pltpu.VMEM_SHARED((NP, width), jnp.float32),
            pltpu.VMEM((8, 128), jnp.int32),
            pltpu.VMEM((8, 128), jnp.int32),
            pltpu.VMEM((128, width), jnp.float32),
            pltpu.VMEM((128, width), jnp.float32),
            pltpu.SemaphoreType.DMA,
            pltpu.SemaphoreType.DMA,
        ],
    )
    return f(tab, src_j, dst_j)


def _sc_final(x2, cfgdst_j, out,
              spart, ci, cio, cbuf, obuf, pbuf, rbuf, sem):
    c = lax.axis_index("c")
    s = lax.axis_index("s")
    coff = c * NP

    pltpu.sync_copy(cfgdst_j.at[s], ci)
    for v in range(4):
        sl = pl.ds(v * 16, 16)
        cio[sl] = ci[sl] + coff
    pltpu.async_copy(x2.at[cio], cbuf, sem).wait()
    cacc = [jnp.zeros((16,), jnp.float32) for _ in range(8)]
    for r in range(64):
        for v in range(8):
            cacc[v] = cacc[v] + cbuf[r, pl.ds(v * 16, 16)]
    for v in range(8):
        pbuf[1, pl.ds(v * 16, 16)] = cacc[v]

    def kbody(k, carry):
        pltpu.sync_copy(x2.at[pl.ds(coff + s * ROWS_T + k * 64, 64)], obuf)
        vs = list(carry)
        for r in range(64):
            for v in range(8):
                vs[v] = vs[v] + obuf[r, pl.ds(v * 16, 16)]
        return tuple(vs)

    oacc = lax.fori_loop(0, ROWS_T // 64, kbody,
                         tuple(jnp.zeros((16,), jnp.float32) for _ in range(8)))
    for v in range(8):
        pbuf[0, pl.ds(v * 16, 16)] = oacc[v]

    pltpu.sync_copy(pbuf, spart.at[s])
    plsc.subcore_barrier()

    @pl.when(s == 0)
    def _():
        pltpu.sync_copy(spart, rbuf)
        for g in range(2):
            for v in range(8):
                t = jnp.zeros((16,), jnp.float32)
                for w in range(16):
                    t = t + rbuf[w, g, pl.ds(v * 16, 16)]
                pbuf[g, pl.ds(v * 16, 16)] = t
        pltpu.sync_copy(pbuf, out.at[c])


def _run_sc_final(x2, cfgdst_j):
    f = pl.kernel(
        _sc_final,
        out_type=jax.ShapeDtypeStruct((2, 2, 128), jnp.float32),
        mesh=_mesh,
        scratch_types=[
            pltpu.VMEM_SHARED((16, 2, 128), jnp.float32),
            pltpu.VMEM((64,), jnp.int32),
            pltpu.VMEM((64,), jnp.int32),
            pltpu.VMEM((64, 128), jnp.float32),
            pltpu.VMEM((64, 128), jnp.float32),
            pltpu.VMEM((2, 128), jnp.float32),
            pltpu.VMEM((16, 2, 128), jnp.float32),
            pltpu.SemaphoreType.DMA,
        ],
    )
    return f(x2, cfgdst_j)


def _tc_prenet(nf_ref, ids_ref, d0_ref, d1_ref, cfg_ref,
               emb_ref, w1_ref, b1_ref, w2_ref, b2_ref,
               ys_ref, csc_ref):
    i = pl.program_id(0)
    rows = i * BN + lax.broadcasted_iota(jnp.int32, (BN, 1), 0)
    msk = (rows < N).astype(jnp.float32)
    deg = 1.0 + d0_ref[...] + d1_ref[...]
    inv = lax.rsqrt(deg)[:, None] * msk

    w1 = w1_ref[...]
    ew = jnp.dot(emb_ref[...], w1[158:190], preferred_element_type=jnp.float32)
    oh = (ids_ref[...][:, None] ==
          lax.broadcasted_iota(jnp.int32, (BN, NUM_OPS), 1)).astype(jnp.float32)
    z = (jnp.dot(nf_ref[...], w1[18:158], preferred_element_type=jnp.float32)
         + jnp.dot(oh, ew, preferred_element_type=jnp.float32) + b1_ref[...])
    w1c = w1[0:18]
    w2 = w2_ref[...]
    b2 = b2_ref[...]
    for h in range(2):
        cfg = cfg_ref[h][:, 0:72]
        csc_ref[h, :, 0:72] = cfg * (100.0 * inv)
        csc_ref[h, :, 72:128] = jnp.zeros((BN, 56), jnp.float32)
        for j in range(4):
            zc = jnp.dot(100.0 * cfg[:, j * 18:(j + 1) * 18], w1c,
                         preferred_element_type=jnp.float32)
            x = _leaky(jnp.dot(_leaky(z + zc), w2,
                               preferred_element_type=jnp.float32) + b2)
            ys_ref[h, :, j * 32:(j + 1) * 32] = x * inv


def _run_tc_prenet(node_feats, op_ids, d0, d1, cfgacc, op_emb, w1, b1, w2, b2):
    return pl.pallas_call(
        _tc_prenet,
        grid=(NBLK,),
        in_specs=[
            pl.BlockSpec((BN, DOP), lambda i: (i, 0)),
            pl.BlockSpec((BN,), lambda i: (i,)),
            pl.BlockSpec((BN,), lambda i: (i,)),
            pl.BlockSpec((BN,), lambda i: (i,)),
            pl.BlockSpec((2, BN, 128), lambda i: (0, i, 0)),
            pl.BlockSpec((NUM_OPS, DEMB), lambda i: (0, 0)),
            pl.BlockSpec((190, H), lambda i: (0, 0)),
            pl.BlockSpec((H,), lambda i: (0,)),
            pl.BlockSpec((H, H), lambda i: (0, 0)),
            pl.BlockSpec((H,), lambda i: (0,)),
        ],
        out_specs=[
            pl.BlockSpec((2, BN, 128), lambda i: (0, i, 0)),
            pl.BlockSpec((2, BN, 128), lambda i: (0, i, 0)),
        ],
        out_shape=[
            jax.ShapeDtypeStruct((2, NP, 128), jnp.float32),
            jax.ShapeDtypeStruct((2, NP, 128), jnp.float32),
        ],
    )(node_feats, op_ids, d0, d1, cfgacc, op_emb, w1, b1, w2, b2)


def _tc_layer(final, ys_ref, sx_ref, sc_ref, d0_ref, d1_ref,
              w1_ref, b1_ref, w2_ref, b2_ref, out_ref):
    i = pl.program_id(0)
    rows = i * BN + lax.broadcasted_iota(jnp.int32, (BN, 1), 0)
    msk = (rows < N).astype(jnp.float32)
    deg = 1.0 + d0_ref[...] + d1_ref[...]
    inv = lax.rsqrt(deg)[:, None]
    sq = jnp.sqrt(deg)[:, None]
    oscale = msk if final else inv * msk

    w1 = w1_ref[...]
    w1c = w1[0:18]
    w1x = w1[18:50]
    b1 = b1_ref[...]
    w2 = w2_ref[...]
    b2 = b2_ref[...]
    for h in range(2):
        x = ys_ref[h] * sq
        aggx = sx_ref[h] * inv
        aggc = sc_ref[h] * inv
        for j in range(4):
            pre = (jnp.dot(aggc[:, j * 18:(j + 1) * 18], w1c,
                           preferred_element_type=jnp.float32)
                   + jnp.dot(aggx[:, j * 32:(j + 1) * 32], w1x,
                             preferred_element_type=jnp.float32) + b1)
            y = _leaky(jnp.dot(_leaky(pre), w2,
                               preferred_element_type=jnp.float32) + b2)
            xn = x[:, j * 32:(j + 1) * 32] + y
            out_ref[h, :, j * 32:(j + 1) * 32] = xn * oscale


def _run_tc_layer(ys, sx, scfg, d0, d1, w1, b1, w2, b2, final):
    return pl.pallas_call(
        functools.partial(_tc_layer, final),
        grid=(NBLK,),
        in_specs=[
            pl.BlockSpec((2, BN, 128), lambda i: (0, i, 0)),
            pl.BlockSpec((2, BN, 128), lambda i: (0, i, 0)),
            pl.BlockSpec((2, BN, 128), lambda i: (0, i, 0)),
            pl.BlockSpec((BN,), lambda i: (i,)),
            pl.BlockSpec((BN,), lambda i: (i,)),
            pl.BlockSpec((50, H), lambda i: (0, 0)),
            pl.BlockSpec((H,), lambda i: (0,)),
            pl.BlockSpec((H, H), lambda i: (0, 0)),
            pl.BlockSpec((H,), lambda i: (0,)),
        ],
        out_specs=[pl.BlockSpec((2, BN, 128), lambda i: (0, i, 0))],
        out_shape=[jax.ShapeDtypeStruct((2, NP, 128), jnp.float32)],
    )(ys, sx, scfg, d0, d1, w1, b1, w2, b2)[0]


def _tc_postnet(op_ref, cfg_ref, w1_ref, w2_ref, out_ref):
    def l2n(v):
        return v * lax.rsqrt(jnp.maximum(jnp.sum(v * v, axis=-1, keepdims=True),
                                         1e-12))

    op_sum = op_ref[...]
    cfg_sum = cfg_ref[...]
    feat = jnp.concatenate([op_sum / float(N), l2n(op_sum), l2n(cfg_sum)],
                           axis=-1)
    r = jnp.dot(_leaky(jnp.dot(feat, w1_ref[...],
                               preferred_element_type=jnp.float32)),
                w2_ref[...], preferred_element_type=jnp.float32)
    out_ref[...] = r


def _run_tc_postnet(op_sum, cfg_sum, w1, w2):
    return pl.pallas_call(
        _tc_postnet,
        out_shape=jax.ShapeDtypeStruct((C, 1), jnp.float32),
    )(op_sum, cfg_sum, w1, w2)


def kernel(node_feats, config_feats, op_emb, prenet_w1, prenet_b1, prenet_w2,
           prenet_b2, gc0_w1, gc0_b1, gc0_w2, gc0_b2, gc1_w1, gc1_b1, gc1_w2,
           gc1_b2, postnet_w1, postnet_w2, op_ids, feed_src, feed_dst,
           cfg_src, cfg_dst):
    del cfg_src

    pad_e = (N + (jnp.arange(EP - E, dtype=jnp.int32) % (NP - N))).astype(jnp.int32)
    src_flat = jnp.concatenate([feed_src, pad_e])
    dst_flat = jnp.concatenate([feed_dst, pad_e])
    src_j = src_flat.reshape(ECH, 128)
    dst_j = dst_flat.reshape(ECH, 128)
    pad_c = (N + (jnp.arange(NCP - NC, dtype=jnp.int32) % (NP - N))).astype(jnp.int32)
    cfgdst_j = jnp.concatenate([cfg_dst, pad_c]).reshape(16, 64)
    cfgf = jnp.concatenate(
        [config_feats.reshape(NC, 2, 72),
         jnp.zeros((NCP - NC, 2, 72), jnp.float32)]).transpose(1, 0, 2)
    cfgf = jnp.concatenate([cfgf, jnp.zeros((2, NCP, 56), jnp.float32)], axis=-1)

    deg2, cfgacc = _run_sc_build(src_j, dst_j, cfgf, cfgdst_j)
    d0, d1 = deg2[0], deg2[1]

    ys0, csc = _run_tc_prenet(node_feats, op_ids, d0, d1, cfgacc, op_emb,
                              prenet_w1, prenet_b1, prenet_w2, prenet_b2)

    s_cfg = _run_sc_agg(csc.reshape(2 * NP, 128), src_j, dst_j, 128)

    sx0 = _run_sc_agg(ys0.reshape(2 * NP, 128), src_j, dst_j, 128)
    ys1 = _run_tc_layer(ys0, sx0, s_cfg, d0, d1, gc0_w1, gc0_b1, gc0_w2,
                        gc0_b2, final=False)

    sx1 = _run_sc_agg(ys1.reshape(2 * NP, 128), src_j, dst_j, 128)
    x2 = _run_tc_layer(ys1, sx1, s_cfg, d0, d1, gc1_w1, gc1_b1, gc1_w2,
                       gc1_b2, final=True)

    sums = _run_sc_final(x2.reshape(2 * NP, 128), cfgdst_j)
    op_sum = sums[:, 0, :].reshape(C, H)
    cfg_sum = sums[:, 1, :].reshape(C, H)

    out = _run_tc_postnet(op_sum, cfg_sum, postnet_w1, postnet_w2)
    return out.reshape(1, C)


# speedup vs baseline: 80.7599x; 1.0164x over previous
import functools

import jax
import jax.numpy as jnp
from jax import lax
from jax.experimental import pallas as pl
from jax.experimental.pallas import tpu as pltpu
from jax.experimental.pallas import tpu_sc as plsc

N = 10000
E = 160000
NC = 1000
C = 8
DOP = 140
DEMB = 32
DCFG = 18
H = 32
NUM_OPS = 120

NP = 10240
EP = 163840
ECH = EP // 128
ECH64 = EP // 64
CH64_T = ECH64 // 16
NCP = 1024
BN = 1024
NBLK = NP // BN
ROWS_T = NP // 16
CH_T = ECH // 16
CH_T2 = ECH // 32

_mesh = plsc.VectorSubcoreMesh(core_axis_name="c", subcore_axis_name="s")


def _leaky(x):
    return jnp.where(x > 0, x, 0.2 * x)


def _sc_build(src_j, dst_j, cfgf, cfgdst_j, zerov, onesv, zer64,
              deg_out, cfg_out,
              sdeg, scfg, vz, vo, sidx, didx, ci, crows):
    c = lax.axis_index("c")
    s = lax.axis_index("s")
    pltpu.sync_copy(zerov, vz)
    pltpu.sync_copy(onesv, vo)
    pltpu.sync_copy(zer64, crows)
    r0 = s * ROWS_T
    for r in range(ROWS_T // 128):
        pltpu.sync_copy(vz, sdeg.at[pl.ds(r0 + r * 128, 128)])
    for r in range(ROWS_T // 64):
        pltpu.sync_copy(crows, scfg.at[pl.ds(r0 + r * 64, 64)])
    plsc.subcore_barrier()

    base = (c * 16 + s) * CH_T2
    pltpu.sync_copy(src_j.at[pl.ds(base, CH_T2)], sidx)
    pltpu.sync_copy(dst_j.at[pl.ds(base, CH_T2)], didx)

    for j in range(CH_T2):
        pltpu.sync_copy(vo, sdeg.at[sidx.at[j]], add=True)
        pltpu.sync_copy(vo, sdeg.at[didx.at[j]], add=True)

    pltpu.sync_copy(cfgdst_j.at[s], ci)
    pltpu.sync_copy(cfgf.at[c, pl.ds(s * 64, 64)], crows)
    pltpu.sync_copy(crows, scfg.at[ci], add=True)
    plsc.subcore_barrier()

    pltpu.sync_copy(sdeg.at[pl.ds(r0, ROWS_T)], deg_out.at[c, pl.ds(r0, ROWS_T)])
    for r in range(ROWS_T // 64):
        pltpu.sync_copy(scfg.at[pl.ds(r0 + r * 64, 64)],
                        cfg_out.at[c, pl.ds(r0 + r * 64, 64)])


def _run_sc_build(src_j, dst_j, cfgf, cfgdst_j):
    zerov = jnp.zeros((128,), jnp.float32)
    onesv = jnp.ones((128,), jnp.float32)
    zer64 = jnp.zeros((64, 128), jnp.float32)
    f = pl.kernel(
        _sc_build,
        out_type=(
            jax.ShapeDtypeStruct((2, NP), jnp.float32),
            jax.ShapeDtypeStruct((2, NP, 128), jnp.float32),
        ),
        mesh=_mesh,
        scratch_types=[
            pltpu.VMEM_SHARED((NP,), jnp.float32),
            pltpu.VMEM_SHARED((NP, 128), jnp.float32),
            pltpu.VMEM((128,), jnp.float32),
            pltpu.VMEM((128,), jnp.float32),
            pltpu.VMEM((CH_T2, 128), jnp.int32),
            pltpu.VMEM((CH_T2, 128), jnp.int32),
            pltpu.VMEM((64,), jnp.int32),
            pltpu.VMEM((64, 128), jnp.float32),
        ],
    )
    return f(src_j, dst_j, cfgf, cfgdst_j, zerov, onesv, zer64)


CPB = 16
NBLK_E = CH64_T // CPB


def _sc_agg(tab, src_j, dst_j, out,
            acc, gj, sj, b0, b1, b2, b3,
            sg0, sg1, sg2, sg3, ss0, ss1, ss2, ss3):
    c = lax.axis_index("c")
    s = lax.axis_index("s")
    coff = c * NP
    r0 = s * ROWS_T
    bufs = [(b0, sg0, ss0), (b1, sg1, ss1), (b2, sg2, ss2), (b3, sg3, ss3)]

    def ibody(r, _):
        pltpu.sync_copy(tab.at[pl.ds(coff + r0 + r * 64, 64)], b0)
        pltpu.sync_copy(b0, acc.at[pl.ds(r0 + r * 64, 64)])
        return 0

    lax.fori_loop(0, ROWS_T // 64, ibody, 0)
    plsc.subcore_barrier()

    def run_dir(gsel, ssel):
        def issue_g(j):
            b, sg, _ = bufs[j % 4]
            pltpu.async_copy(tab.at[gj.at[j]], b, sg)

        def wait_g(j):
            b, sg, _ = bufs[j % 4]
            pltpu.make_async_copy(tab.at[gj.at[j]], b, sg).wait()

        def start_s(j):
            b, _, ss = bufs[j % 4]
            pltpu.async_copy(b, acc.at[sj.at[j]], ss, add=True)

        def wait_s(j):
            b, _, ss = bufs[j % 4]
            pltpu.make_async_copy(b, acc.at[sj.at[j]], ss).wait()

        def block(b, _):
            row0 = s * CH64_T + b * CPB
            pltpu.sync_copy(gsel.at[pl.ds(row0, CPB)], gj)
            pltpu.sync_copy(ssel.at[pl.ds(row0, CPB)], sj)
            for r in range(CPB):
                for v in range(4):
                    sl = pl.ds(v * 16, 16)
                    gj[r, sl] = gj[r, sl] + coff
            issue_g(0)
            issue_g(1)
            for j in range(CPB):
                if j >= 2:
                    wait_s(j - 2)
                if j + 2 < CPB:
                    issue_g(j + 2)
                wait_g(j)
                start_s(j)
            wait_s(CPB - 2)
            wait_s(CPB - 1)
            return 0

        lax.fori_loop(0, NBLK_E, block, 0)

    run_dir(src_j, dst_j)
    run_dir(dst_j, src_j)
    plsc.subcore_barrier()

    def wbody(r, _):
        pltpu.sync_copy(acc.at[pl.ds(r0 + r * 64, 64)], b1)
        pltpu.sync_copy(b1, out.at[c, pl.ds(r0 + r * 64, 64)])
        return 0

    lax.fori_loop(0, ROWS_T // 64, wbody, 0)


def _run_sc_agg(tab, src_j, dst_j, width):
    f = pl.kernel(
        _sc_agg,
        out_type=jax.ShapeDtypeStruct((2, NP, width), jnp.float32),
        mesh=_mesh,
        scratch_types=[
            pltpu.VMEM_SHARED((NP, width), jnp.float32),
            pltpu.VMEM((CPB, 64), jnp.int32),
            pltpu.VMEM((CPB, 64), jnp.int32),
            pltpu.VMEM((64, width), jnp.float32),
            pltpu.VMEM((64, width), jnp.float32),
            pltpu.VMEM((64, width), jnp.float32),
            pltpu.VMEM((64, width), jnp.float32),
            pltpu.SemaphoreType.DMA,
            pltpu.SemaphoreType.DMA,
            pltpu.SemaphoreType.DMA,
            pltpu.SemaphoreType.DMA,
            pltpu.SemaphoreType.DMA,
            pltpu.SemaphoreType.DMA,
            pltpu.SemaphoreType.DMA,
            pltpu.SemaphoreType.DMA,
        ],
    )
    return f(tab, src_j, dst_j)


def _sc_final(x2, cfgdst_j, out,
              spart, ci, cio, cbuf, obuf, pbuf, rbuf, sem):
    c = lax.axis_index("c")
    s = lax.axis_index("s")
    coff = c * NP

    pltpu.sync_copy(cfgdst_j.at[s], ci)
    for v in range(4):
        sl = pl.ds(v * 16, 16)
        cio[sl] = ci[sl] + coff
    pltpu.async_copy(x2.at[cio], cbuf, sem).wait()
    cacc = [jnp.zeros((16,), jnp.float32) for _ in range(8)]
    for r in range(64):
        for v in range(8):
            cacc[v] = cacc[v] + cbuf[r, pl.ds(v * 16, 16)]
    for v in range(8):
        pbuf[1, pl.ds(v * 16, 16)] = cacc[v]

    def kbody(k, carry):
        pltpu.sync_copy(x2.at[pl.ds(coff + s * ROWS_T + k * 64, 64)], obuf)
        vs = list(carry)
        for r in range(64):
            for v in range(8):
                vs[v] = vs[v] + obuf[r, pl.ds(v * 16, 16)]
        return tuple(vs)

    oacc = lax.fori_loop(0, ROWS_T // 64, kbody,
                         tuple(jnp.zeros((16,), jnp.float32) for _ in range(8)))
    for v in range(8):
        pbuf[0, pl.ds(v * 16, 16)] = oacc[v]

    pltpu.sync_copy(pbuf, spart.at[s])
    plsc.subcore_barrier()

    @pl.when(s == 0)
    def _():
        pltpu.sync_copy(spart, rbuf)
        for g in range(2):
            for v in range(8):
                t = jnp.zeros((16,), jnp.float32)
                for w in range(16):
                    t = t + rbuf[w, g, pl.ds(v * 16, 16)]
                pbuf[g, pl.ds(v * 16, 16)] = t
        pltpu.sync_copy(pbuf, out.at[c])


def _run_sc_final(x2, cfgdst_j):
    f = pl.kernel(
        _sc_final,
        out_type=jax.ShapeDtypeStruct((2, 2, 128), jnp.float32),
        mesh=_mesh,
        scratch_types=[
            pltpu.VMEM_SHARED((16, 2, 128), jnp.float32),
            pltpu.VMEM((64,), jnp.int32),
            pltpu.VMEM((64,), jnp.int32),
            pltpu.VMEM((64, 128), jnp.float32),
            pltpu.VMEM((64, 128), jnp.float32),
            pltpu.VMEM((2, 128), jnp.float32),
            pltpu.VMEM((16, 2, 128), jnp.float32),
            pltpu.SemaphoreType.DMA,
        ],
    )
    return f(x2, cfgdst_j)


def _tc_prenet(nf_ref, ids_ref, d0_ref, d1_ref, cfg_ref,
               emb_ref, w1_ref, b1_ref, w2_ref, b2_ref,
               ys_ref, csc_ref):
    i = pl.program_id(0)
    rows = i * BN + lax.broadcasted_iota(jnp.int32, (BN, 1), 0)
    msk = (rows < N).astype(jnp.float32)
    deg = 1.0 + d0_ref[...] + d1_ref[...]
    inv = lax.rsqrt(deg)[:, None] * msk

    w1 = w1_ref[...]
    ew = jnp.dot(emb_ref[...], w1[158:190], preferred_element_type=jnp.float32)
    oh = (ids_ref[...][:, None] ==
          lax.broadcasted_iota(jnp.int32, (BN, NUM_OPS), 1)).astype(jnp.float32)
    z = (jnp.dot(nf_ref[...], w1[18:158], preferred_element_type=jnp.float32)
         + jnp.dot(oh, ew, preferred_element_type=jnp.float32) + b1_ref[...])
    w1c = w1[0:18]
    w2 = w2_ref[...]
    b2 = b2_ref[...]
    for h in range(2):
        cfg = cfg_ref[h][:, 0:72]
        csc_ref[h, :, 0:72] = cfg * (100.0 * inv)
        csc_ref[h, :, 72:128] = jnp.zeros((BN, 56), jnp.float32)
        for j in range(4):
            zc = jnp.dot(100.0 * cfg[:, j * 18:(j + 1) * 18], w1c,
                         preferred_element_type=jnp.float32)
            x = _leaky(jnp.dot(_leaky(z + zc), w2,
                               preferred_element_type=jnp.float32) + b2)
            ys_ref[h, :, j * 32:(j + 1) * 32] = x * inv


def _run_tc_prenet(node_feats, op_ids, d0, d1, cfgacc, op_emb, w1, b1, w2, b2):
    return pl.pallas_call(
        _tc_prenet,
        grid=(NBLK,),
        in_specs=[
            pl.BlockSpec((BN, DOP), lambda i: (i, 0)),
            pl.BlockSpec((BN,), lambda i: (i,)),
            pl.BlockSpec((BN,), lambda i: (i,)),
            pl.BlockSpec((BN,), lambda i: (i,)),
            pl.BlockSpec((2, BN, 128), lambda i: (0, i, 0)),
            pl.BlockSpec((NUM_OPS, DEMB), lambda i: (0, 0)),
            pl.BlockSpec((190, H), lambda i: (0, 0)),
            pl.BlockSpec((H,), lambda i: (0,)),
            pl.BlockSpec((H, H), lambda i: (0, 0)),
            pl.BlockSpec((H,), lambda i: (0,)),
        ],
        out_specs=[
            pl.BlockSpec((2, BN, 128), lambda i: (0, i, 0)),
            pl.BlockSpec((2, BN, 128), lambda i: (0, i, 0)),
        ],
        out_shape=[
            jax.ShapeDtypeStruct((2, NP, 128), jnp.float32),
            jax.ShapeDtypeStruct((2, NP, 128), jnp.float32),
        ],
    )(node_feats, op_ids, d0, d1, cfgacc, op_emb, w1, b1, w2, b2)


def _tc_layer(final, ys_ref, sx_ref, sc_ref, d0_ref, d1_ref,
              w1_ref, b1_ref, w2_ref, b2_ref, out_ref):
    i = pl.program_id(0)
    rows = i * BN + lax.broadcasted_iota(jnp.int32, (BN, 1), 0)
    msk = (rows < N).astype(jnp.float32)
    deg = 1.0 + d0_ref[...] + d1_ref[...]
    inv = lax.rsqrt(deg)[:, None]
    sq = jnp.sqrt(deg)[:, None]
    oscale = msk if final else inv * msk

    w1 = w1_ref[...]
    w1c = w1[0:18]
    w1x = w1[18:50]
    b1 = b1_ref[...]
    w2 = w2_ref[...]
    b2 = b2_ref[...]
    for h in range(2):
        x = ys_ref[h] * sq
        aggx = sx_ref[h] * inv
        aggc = sc_ref[h] * inv
        for j in range(4):
            pre = (jnp.dot(aggc[:, j * 18:(j + 1) * 18], w1c,
                           preferred_element_type=jnp.float32)
                   + jnp.dot(aggx[:, j * 32:(j + 1) * 32], w1x,
                             preferred_element_type=jnp.float32) + b1)
            y = _leaky(jnp.dot(_leaky(pre), w2,
                               preferred_element_type=jnp.float32) + b2)
            xn = x[:, j * 32:(j + 1) * 32] + y
            out_ref[h, :, j * 32:(j + 1) * 32] = xn * oscale


def _run_tc_layer(ys, sx, scfg, d0, d1, w1, b1, w2, b2, final):
    return pl.pallas_call(
        functools.partial(_tc_layer, final),
        grid=(NBLK,),
        in_specs=[
            pl.BlockSpec((2, BN, 128), lambda i: (0, i, 0)),
            pl.BlockSpec((2, BN, 128), lambda i: (0, i, 0)),
            pl.BlockSpec((2, BN, 128), lambda i: (0, i, 0)),
            pl.BlockSpec((BN,), lambda i: (i,)),
            pl.BlockSpec((BN,), lambda i: (i,)),
            pl.BlockSpec((50, H), lambda i: (0, 0)),
            pl.BlockSpec((H,), lambda i: (0,)),
            pl.BlockSpec((H, H), lambda i: (0, 0)),
            pl.BlockSpec((H,), lambda i: (0,)),
        ],
        out_specs=[pl.BlockSpec((2, BN, 128), lambda i: (0, i, 0))],
        out_shape=[jax.ShapeDtypeStruct((2, NP, 128), jnp.float32)],
    )(ys, sx, scfg, d0, d1, w1, b1, w2, b2)[0]


def _tc_postnet(op_ref, cfg_ref, w1_ref, w2_ref, out_ref):
    def l2n(v):
        return v * lax.rsqrt(jnp.maximum(jnp.sum(v * v, axis=-1, keepdims=True),
                                         1e-12))

    op_sum = op_ref[...]
    cfg_sum = cfg_ref[...]
    feat = jnp.concatenate([op_sum / float(N), l2n(op_sum), l2n(cfg_sum)],
                           axis=-1)
    r = jnp.dot(_leaky(jnp.dot(feat, w1_ref[...],
                               preferred_element_type=jnp.float32)),
                w2_ref[...], preferred_element_type=jnp.float32)
    out_ref[...] = r


def _run_tc_postnet(op_sum, cfg_sum, w1, w2):
    return pl.pallas_call(
        _tc_postnet,
        out_shape=jax.ShapeDtypeStruct((C, 1), jnp.float32),
    )(op_sum, cfg_sum, w1, w2)


def kernel(node_feats, config_feats, op_emb, prenet_w1, prenet_b1, prenet_w2,
           prenet_b2, gc0_w1, gc0_b1, gc0_w2, gc0_b2, gc1_w1, gc1_b1, gc1_w2,
           gc1_b2, postnet_w1, postnet_w2, op_ids, feed_src, feed_dst,
           cfg_src, cfg_dst):
    del cfg_src

    pad_e = (N + (jnp.arange(EP - E, dtype=jnp.int32) % (NP - N))).astype(jnp.int32)
    src_flat = jnp.concatenate([feed_src, pad_e])
    dst_flat = jnp.concatenate([feed_dst, pad_e])
    src_j = src_flat.reshape(ECH, 128)
    dst_j = dst_flat.reshape(ECH, 128)
    src_j64 = src_flat.reshape(ECH64, 64)
    dst_j64 = dst_flat.reshape(ECH64, 64)
    pad_c = (N + (jnp.arange(NCP - NC, dtype=jnp.int32) % (NP - N))).astype(jnp.int32)
    cfgdst_j = jnp.concatenate([cfg_dst, pad_c]).reshape(16, 64)
    cfgf = jnp.concatenate(
        [config_feats.reshape(NC, 2, 72),
         jnp.zeros((NCP - NC, 2, 72), jnp.float32)]).transpose(1, 0, 2)
    cfgf = jnp.concatenate([cfgf, jnp.zeros((2, NCP, 56), jnp.float32)], axis=-1)

    deg2, cfgacc = _run_sc_build(src_j, dst_j, cfgf, cfgdst_j)
    d0, d1 = deg2[0], deg2[1]

    ys0, csc = _run_tc_prenet(node_feats, op_ids, d0, d1, cfgacc, op_emb,
                              prenet_w1, prenet_b1, prenet_w2, prenet_b2)

    s_cfg = _run_sc_agg(csc.reshape(2 * NP, 128), src_j64, dst_j64, 128)

    sx0 = _run_sc_agg(ys0.reshape(2 * NP, 128), src_j64, dst_j64, 128)
    ys1 = _run_tc_layer(ys0, sx0, s_cfg, d0, d1, gc0_w1, gc0_b1, gc0_w2,
                        gc0_b2, final=False)

    sx1 = _run_sc_agg(ys1.reshape(2 * NP, 128), src_j64, dst_j64, 128)
    x2 = _run_tc_layer(ys1, sx1, s_cfg, d0, d1, gc1_w1, gc1_b1, gc1_w2,
                       gc1_b2, final=True)

    sums = _run_sc_final(x2.reshape(2 * NP, 128), cfgdst_j)
    op_sum = sums[:, 0, :].reshape(C, H)
    cfg_sum = sums[:, 1, :].reshape(C, H)

    out = _run_tc_postnet(op_sum, cfg_sum, postnet_w1, postnet_w2)
    return out.reshape(1, C)
